# baseline (speedup 1.0000x reference)
import functools

import jax
import jax.numpy as jnp
from jax import lax
from jax.experimental import pallas as pl
from jax.experimental.pallas import tpu as pltpu
from jax.experimental.pallas import tpu_sc as plsc

_NUM_STEPS = 1000
_DIM = 128
_BATCH = 16384
_CHUNK = 128


def _mlp_body(emb_ref, w1_ref, b1_ref, w2_ref, b2_ref, out_ref):
    x = emb_ref[...]
    h = lax.dot_general(x, w1_ref[...], (((1,), (1,)), ((), ())),
                        preferred_element_type=jnp.float32) + b1_ref[...]
    h = h * (1.0 / (1.0 + jnp.exp(-h)))
    h = lax.dot_general(h, w2_ref[...], (((1,), (1,)), ((), ())),
                        preferred_element_type=jnp.float32) + b2_ref[...]
    out_ref[...] = h * (1.0 / (1.0 + jnp.exp(-h)))


@jax.jit
def _mlp_table(embedding, W1, b1, W2, b2):
    return pl.pallas_call(
        _mlp_body,
        out_shape=jax.ShapeDtypeStruct((_NUM_STEPS, _DIM), jnp.float32),
    )(embedding, W1, b1.reshape(1, _DIM), W2, b2.reshape(1, _DIM))


@functools.cache
def _make_gather(num_cores, num_subcores):
    nw = num_cores * num_subcores
    b_per_w = _BATCH // nw
    chunks = b_per_w // _CHUNK
    mesh = plsc.VectorSubcoreMesh(core_axis_name="c", subcore_axis_name="s",
                                  num_cores=num_cores,
                                  num_subcores=num_subcores)

    @functools.partial(
        pl.kernel,
        mesh=mesh,
        out_type=jax.ShapeDtypeStruct((_BATCH, _DIM), jnp.float32),
        scratch_types=[
            pltpu.VMEM((chunks, _CHUNK), jnp.int32),
            pltpu.VMEM((b_per_w, _DIM), jnp.float32),
            pltpu.SemaphoreType.DMA((chunks,)),
            pltpu.SemaphoreType.DMA,
        ],
    )
    def gather_kernel(table_hbm, idx_hbm, out_hbm, idx_v, rows_v, gsem, wsem):
        wid = lax.axis_index("s") * num_cores + lax.axis_index("c")
        pltpu.sync_copy(idx_hbm.at[pl.ds(wid * chunks, chunks)], idx_v)
        gathers = [
            pltpu.async_copy(table_hbm.at[idx_v.at[j]],
                             rows_v.at[pl.ds(j * _CHUNK, _CHUNK)],
                             gsem.at[j])
            for j in range(chunks)
        ]
        writes = []
        for j in range(chunks):
            gathers[j].wait()
            writes.append(
                pltpu.async_copy(rows_v.at[pl.ds(j * _CHUNK, _CHUNK)],
                                 out_hbm.at[pl.ds(wid * b_per_w + j * _CHUNK,
                                                  _CHUNK)],
                                 wsem))
        for c in writes:
            c.wait()

    return gather_kernel


def kernel(diffusion_step, embedding, W1, b1, W2, b2):
    table = _mlp_table(embedding, W1, b1, W2, b2)
    info = plsc.get_sparse_core_info()
    gather = _make_gather(info.num_cores, info.num_subcores)
    idx = diffusion_step.astype(jnp.int32).reshape(-1, _CHUNK)
    return gather(table, idx)


# speedup vs baseline: 1.0525x; 1.0525x over previous
import functools

import jax
import jax.numpy as jnp
from jax import lax
from jax.experimental import pallas as pl
from jax.experimental.pallas import tpu as pltpu
from jax.experimental.pallas import tpu_sc as plsc

_NUM_STEPS = 1000
_DIM = 128
_BATCH = 16384
_CHUNK = 128


def _mlp_body(emb_ref, w1_ref, b1_ref, w2_ref, b2_ref, out_ref):
    x = emb_ref[...]
    h = lax.dot_general(x, w1_ref[...], (((1,), (1,)), ((), ())),
                        preferred_element_type=jnp.float32) + b1_ref[...]
    h = h * (1.0 / (1.0 + jnp.exp(-h)))
    h = lax.dot_general(h, w2_ref[...], (((1,), (1,)), ((), ())),
                        preferred_element_type=jnp.float32) + b2_ref[...]
    out_ref[...] = h * (1.0 / (1.0 + jnp.exp(-h)))


@jax.jit
def _mlp_table(embedding, W1, b1, W2, b2):
    return pl.pallas_call(
        _mlp_body,
        out_shape=jax.ShapeDtypeStruct((_NUM_STEPS, _DIM), jnp.float32),
    )(embedding, W1, b1.reshape(1, _DIM), W2, b2.reshape(1, _DIM))


@functools.cache
def _make_gather(num_cores, num_subcores):
    nw = num_cores * num_subcores
    b_per_w = _BATCH // nw
    chunks = b_per_w // _CHUNK
    mesh = plsc.VectorSubcoreMesh(core_axis_name="c", subcore_axis_name="s",
                                  num_cores=num_cores,
                                  num_subcores=num_subcores)

    @functools.partial(
        pl.kernel,
        mesh=mesh,
        out_type=jax.ShapeDtypeStruct((_BATCH, _DIM), jnp.float32),
        scratch_types=[
            pltpu.VMEM((b_per_w,), jnp.int32),
            pltpu.VMEM((b_per_w, _DIM), jnp.float32),
            pltpu.SemaphoreType.DMA,
        ],
    )
    def gather_kernel(table_hbm, idx_hbm, out_hbm, idx_v, rows_v, sem):
        wid = lax.axis_index("s") * num_cores + lax.axis_index("c")
        pltpu.sync_copy(idx_hbm.at[pl.ds(wid * b_per_w, b_per_w)], idx_v)
        copies = [
            pltpu.async_copy(table_hbm.at[idx_v.at[pl.ds(j * _CHUNK, _CHUNK)]],
                             rows_v.at[pl.ds(j * _CHUNK, _CHUNK)], sem)
            for j in range(chunks)
        ]
        for c in copies:
            c.wait()
        pltpu.sync_copy(rows_v, out_hbm.at[pl.ds(wid * b_per_w, b_per_w)])

    return gather_kernel


def kernel(diffusion_step, embedding, W1, b1, W2, b2):
    table = _mlp_table(embedding, W1, b1, W2, b2)
    info = plsc.get_sparse_core_info()
    gather = _make_gather(info.num_cores, info.num_subcores)
    return gather(table, diffusion_step)


# speedup vs baseline: 1.1158x; 1.0602x over previous
import functools

import jax
import jax.numpy as jnp
from jax import lax
from jax.experimental import pallas as pl
from jax.experimental.pallas import tpu as pltpu
from jax.experimental.pallas import tpu_sc as plsc

_NUM_STEPS = 1000
_DIM = 128
_BATCH = 16384
_CHUNK = 128


def _mlp_body(emb_ref, w1_ref, b1_ref, w2_ref, b2_ref, out_ref):
    x = emb_ref[...]
    h = lax.dot_general(x, w1_ref[...], (((1,), (1,)), ((), ())),
                        preferred_element_type=jnp.float32) + b1_ref[...]
    h = h * (1.0 / (1.0 + jnp.exp(-h)))
    h = lax.dot_general(h, w2_ref[...], (((1,), (1,)), ((), ())),
                        preferred_element_type=jnp.float32) + b2_ref[...]
    out_ref[...] = h * (1.0 / (1.0 + jnp.exp(-h)))


@jax.jit
def _mlp_table(embedding, W1, b1, W2, b2):
    return pl.pallas_call(
        _mlp_body,
        out_shape=jax.ShapeDtypeStruct((_NUM_STEPS, _DIM), jnp.float32),
    )(embedding, W1, b1.reshape(1, _DIM), W2, b2.reshape(1, _DIM))


@functools.cache
def _make_gather(num_cores, num_subcores):
    nw = num_cores * num_subcores
    b_per_w = _BATCH // nw
    chunks = b_per_w // _CHUNK
    mesh = plsc.VectorSubcoreMesh(core_axis_name="c", subcore_axis_name="s",
                                  num_cores=num_cores,
                                  num_subcores=num_subcores)

    @functools.partial(
        pl.kernel,
        mesh=mesh,
        out_type=jax.ShapeDtypeStruct((_BATCH, _DIM), jnp.float32),
        scratch_types=[
            pltpu.VMEM((b_per_w,), jnp.int32),
            pltpu.VMEM((b_per_w, _DIM), jnp.float32),
            pltpu.VMEM_SHARED((_NUM_STEPS, _DIM), jnp.float32),
            pltpu.SemaphoreType.DMA,
        ],
    )
    def gather_kernel(table_hbm, idx_hbm, out_hbm, idx_v, rows_v, table_sh,
                      sem):
        wid = lax.axis_index("s") * num_cores + lax.axis_index("c")
        pltpu.sync_copy(idx_hbm.at[pl.ds(wid * b_per_w, b_per_w)], idx_v)
        @pl.when(lax.axis_index("s") == 0)
        def _():
            pltpu.sync_copy(table_hbm, table_sh)
        plsc.subcore_barrier()
        copies = [
            pltpu.async_copy(table_sh.at[idx_v.at[pl.ds(j * _CHUNK, _CHUNK)]],
                             rows_v.at[pl.ds(j * _CHUNK, _CHUNK)], sem)
            for j in range(chunks)
        ]
        for c in copies:
            c.wait()
        pltpu.sync_copy(rows_v, out_hbm.at[pl.ds(wid * b_per_w, b_per_w)])

    return gather_kernel


def kernel(diffusion_step, embedding, W1, b1, W2, b2):
    table = _mlp_table(embedding, W1, b1, W2, b2)
    info = plsc.get_sparse_core_info()
    gather = _make_gather(info.num_cores, info.num_subcores)
    return gather(table, diffusion_step)


# speedup vs baseline: 1.1620x; 1.0414x over previous
import functools

import jax
import jax.numpy as jnp
from jax import lax
from jax.experimental import pallas as pl
from jax.experimental.pallas import tpu as pltpu
from jax.experimental.pallas import tpu_sc as plsc

_NUM_STEPS = 1000
_DIM = 128
_BATCH = 16384
_CHUNK = 128


def _mlp_body(emb_ref, w1_ref, b1_ref, w2_ref, b2_ref, out_ref):
    x = emb_ref[...]
    h = lax.dot_general(x, w1_ref[...], (((1,), (1,)), ((), ())),
                        preferred_element_type=jnp.float32) + b1_ref[...]
    h = h * (1.0 / (1.0 + jnp.exp(-h)))
    h = lax.dot_general(h, w2_ref[...], (((1,), (1,)), ((), ())),
                        preferred_element_type=jnp.float32) + b2_ref[...]
    out_ref[...] = h * (1.0 / (1.0 + jnp.exp(-h)))


@jax.jit
def _mlp_table(embedding, W1, b1, W2, b2):
    return pl.pallas_call(
        _mlp_body,
        out_shape=jax.ShapeDtypeStruct((_NUM_STEPS, _DIM), jnp.float32),
    )(embedding, W1, b1.reshape(1, _DIM), W2, b2.reshape(1, _DIM))


@functools.cache
def _make_gather(num_cores, num_subcores):
    nw = num_cores * num_subcores
    b_per_w = _BATCH // nw
    chunks = b_per_w // _CHUNK
    mesh = plsc.VectorSubcoreMesh(core_axis_name="c", subcore_axis_name="s",
                                  num_cores=num_cores,
                                  num_subcores=num_subcores)

    @functools.partial(
        pl.kernel,
        mesh=mesh,
        out_type=jax.ShapeDtypeStruct((_BATCH, _DIM), jnp.float32),
        scratch_types=[
            pltpu.VMEM((b_per_w,), jnp.int32),
            pltpu.VMEM((b_per_w, _DIM), jnp.float32),
            pltpu.VMEM_SHARED((_NUM_STEPS, _DIM), jnp.float32),
            pltpu.SemaphoreType.DMA((chunks,)),
            pltpu.SemaphoreType.DMA,
        ],
    )
    def gather_kernel(table_hbm, idx_hbm, out_hbm, idx_v, rows_v, table_sh,
                      gsem, wsem):
        wid = lax.axis_index("s") * num_cores + lax.axis_index("c")
        pltpu.sync_copy(idx_hbm.at[pl.ds(wid * b_per_w, b_per_w)], idx_v)
        @pl.when(lax.axis_index("s") == 0)
        def _():
            pltpu.sync_copy(table_hbm, table_sh)
        plsc.subcore_barrier()
        gathers = [
            pltpu.async_copy(table_sh.at[idx_v.at[pl.ds(j * _CHUNK, _CHUNK)]],
                             rows_v.at[pl.ds(j * _CHUNK, _CHUNK)],
                             gsem.at[j])
            for j in range(chunks)
        ]
        writes = []
        for j in range(chunks):
            gathers[j].wait()
            writes.append(
                pltpu.async_copy(
                    rows_v.at[pl.ds(j * _CHUNK, _CHUNK)],
                    out_hbm.at[pl.ds(wid * b_per_w + j * _CHUNK, _CHUNK)],
                    wsem))
        for c in writes:
            c.wait()

    return gather_kernel


def kernel(diffusion_step, embedding, W1, b1, W2, b2):
    table = _mlp_table(embedding, W1, b1, W2, b2)
    info = plsc.get_sparse_core_info()
    gather = _make_gather(info.num_cores, info.num_subcores)
    return gather(table, diffusion_step)
